# baseline (speedup 1.0000x reference)
import functools

import jax
import jax.numpy as jnp
from jax import lax
from jax.experimental import pallas as pl
from jax.experimental.pallas import tpu as pltpu
from jax.experimental.pallas import tpu_sc as plsc

NUM_HEADS = 16
T = 1569
K = 10938
KPAD = 10944
FLAT = NUM_HEADS * KPAD

NC, NS, L = 2, 16, 16

HG = 8
NG = NUM_HEADS // HG
NSLICE = (NC * NS) // NG

R = 8
NBLK = (T - 1) // R
NCHUNK = 12
REM0 = NSLICE * NCHUNK * R
NVEC = (T - 1) // L
TAIL = T - L


def _gather_row(tbl_v, idx_v, val_v, r):
  @plsc.parallel_loop(0, NVEC * L, step=L, unroll=7)
  def inner(off):
    iv = idx_v[r, pl.ds(off, L)]
    for hl in range(HG):
      val_v[0, hl, pl.ds(off, L)] = plsc.load_gather(tbl_v, [iv + hl * KPAD])

  iv = idx_v[r, pl.ds(TAIL, L)]
  for hl in range(HG):
    val_v[0, hl, pl.ds(TAIL, L)] = plsc.load_gather(tbl_v, [iv + hl * KPAD])


def _tec_body(tbl_hbm, idx_hbm, out_hbm, tbl_v, idx_v, val0, val1, sv0, sv1):
  wid = lax.axis_index("s") * NC + lax.axis_index("c")
  g = wid % NG
  sl = wid // NG
  h0 = HG * g

  for hl in range(HG):
    pltpu.sync_copy(tbl_hbm.at[pl.ds((h0 + hl) * KPAD, KPAD)],
                    tbl_v.at[pl.ds(hl * KPAD, KPAD)])

  blk0 = NCHUNK * sl

  def block(blk, carry):
    base = (blk0 + blk) * R
    pltpu.sync_copy(idx_hbm.at[pl.ds(base, R), :], idx_v)
    for r in range(R):
      val_v = (val0, val1)[r % 2]
      sem = (sv0, sv1)[r % 2]
      dst = out_hbm.at[pl.ds(base + r, 1), pl.ds(h0, HG), :]
      if r < 2:

        @pl.when(blk >= 1)
        def _():
          pltpu.make_async_copy(val_v, dst, sem).wait()
      else:
        pltpu.make_async_copy(val_v, dst, sem).wait()

      _gather_row(tbl_v, idx_v, val_v, r)
      pltpu.async_copy(val_v, dst, sem)
    return carry

  lax.fori_loop(0, NCHUNK, block, 0)

  for r in (6, 7):
    row = (blk0 + NCHUNK - 1) * R + r
    pltpu.make_async_copy(
        (val0, val1)[r % 2],
        out_hbm.at[pl.ds(row, 1), pl.ds(h0, HG), :],
        (sv0, sv1)[r % 2]).wait()

  def one_row(row):
    pltpu.sync_copy(idx_hbm.at[pl.ds((row // R) * R, R), :], idx_v)
    _gather_row(tbl_v, idx_v, val0, row % R)
    pltpu.sync_copy(val0, out_hbm.at[pl.ds(row, 1), pl.ds(h0, HG), :])

  one_row(REM0 + sl)
  one_row(REM0 + NSLICE + sl)

  @pl.when(sl == NSLICE - 1)
  def _():
    pltpu.sync_copy(idx_hbm.at[pl.ds(NBLK * R, 1), :],
                    idx_v.at[pl.ds(0, 1), :])
    _gather_row(tbl_v, idx_v, val0, 0)
    pltpu.sync_copy(val0, out_hbm.at[pl.ds(NBLK * R, 1), pl.ds(h0, HG), :])


_rpb_call = functools.partial(
    pl.kernel,
    out_type=jax.ShapeDtypeStruct((T, NUM_HEADS, T), jnp.float32),
    mesh=plsc.VectorSubcoreMesh(core_axis_name="c", subcore_axis_name="s"),
    scratch_types=[
        pltpu.VMEM((HG * KPAD,), jnp.float32),
        pltpu.VMEM((R, T), jnp.int32),
        pltpu.VMEM((1, HG, T), jnp.float32),
        pltpu.VMEM((1, HG, T), jnp.float32),
        pltpu.SemaphoreType.DMA,
        pltpu.SemaphoreType.DMA,
    ],
    compiler_params=pltpu.CompilerParams(needs_layout_passes=False),
)(_tec_body)


@jax.jit
def kernel(relative_position_bias_table, relative_position_index):
  tbl = relative_position_bias_table.astype(jnp.float32)
  tbl_flat = jnp.pad(tbl, ((0, KPAD - K), (0, 0))).T.reshape(-1)
  idx = relative_position_index.astype(jnp.int32)
  out3 = _rpb_call(tbl_flat, idx)
  return jnp.transpose(out3, (1, 0, 2))


# speedup vs baseline: 1.0067x; 1.0067x over previous
import functools

import jax
import jax.numpy as jnp
from jax import lax
from jax.experimental import pallas as pl
from jax.experimental.pallas import tpu as pltpu
from jax.experimental.pallas import tpu_sc as plsc

NUM_HEADS = 16
T = 1569
K = 10938
KPAD = 10944
FLAT = NUM_HEADS * KPAD

NC, NS, L = 2, 16, 16

HG = 8
NG = NUM_HEADS // HG
NSLICE = (NC * NS) // NG

R = 8
NBLK = (T - 1) // R
NVEC = (T - 1) // L
TAIL = T - L


def _gather_row(tbl_v, idx_v, val_v, r):
  @plsc.parallel_loop(0, NVEC * L, step=L, unroll=7)
  def inner(off):
    iv = idx_v[r, pl.ds(off, L)]
    for hl in range(HG):
      val_v[0, hl, pl.ds(off, L)] = plsc.load_gather(tbl_v, [iv + hl * KPAD])

  iv = idx_v[r, pl.ds(TAIL, L)]
  for hl in range(HG):
    val_v[0, hl, pl.ds(TAIL, L)] = plsc.load_gather(tbl_v, [iv + hl * KPAD])


def _tec_body(tbl_hbm, idx_hbm, out_hbm, tbl_v, idx_v, val0, val1, sv0, sv1):
  wid = lax.axis_index("s") * NC + lax.axis_index("c")
  g = wid % NG
  sl = wid // NG
  h0 = HG * g

  for hl in range(HG):
    pltpu.sync_copy(tbl_hbm.at[pl.ds((h0 + hl) * KPAD, KPAD)],
                    tbl_v.at[pl.ds(hl * KPAD, KPAD)])

  blk0 = 12 * sl + jnp.minimum(sl, 4)
  nblocks = jnp.where(sl < 4, 13, 12)

  def block(blk, carry):
    base = (blk0 + blk) * R
    pltpu.sync_copy(idx_hbm.at[pl.ds(base, R), :], idx_v)
    for r in range(R):
      val_v = (val0, val1)[r % 2]
      sem = (sv0, sv1)[r % 2]
      dst = out_hbm.at[pl.ds(base + r, 1), pl.ds(h0, HG), :]
      if r < 2:

        @pl.when(blk >= 1)
        def _():
          pltpu.make_async_copy(val_v, dst, sem).wait()
      else:
        pltpu.make_async_copy(val_v, dst, sem).wait()

      _gather_row(tbl_v, idx_v, val_v, r)
      pltpu.async_copy(val_v, dst, sem)
    return carry

  lax.fori_loop(0, nblocks, block, 0)

  for r in (6, 7):
    row = (blk0 + nblocks - 1) * R + r
    pltpu.make_async_copy(
        (val0, val1)[r % 2],
        out_hbm.at[pl.ds(row, 1), pl.ds(h0, HG), :],
        (sv0, sv1)[r % 2]).wait()

  @pl.when(sl == NSLICE - 1)
  def _():
    pltpu.sync_copy(idx_hbm.at[pl.ds(NBLK * R, 1), :],
                    idx_v.at[pl.ds(0, 1), :])
    _gather_row(tbl_v, idx_v, val0, 0)
    pltpu.sync_copy(val0, out_hbm.at[pl.ds(NBLK * R, 1), pl.ds(h0, HG), :])


_rpb_call = functools.partial(
    pl.kernel,
    out_type=jax.ShapeDtypeStruct((T, NUM_HEADS, T), jnp.float32),
    mesh=plsc.VectorSubcoreMesh(core_axis_name="c", subcore_axis_name="s"),
    scratch_types=[
        pltpu.VMEM((HG * KPAD,), jnp.float32),
        pltpu.VMEM((R, T), jnp.int32),
        pltpu.VMEM((1, HG, T), jnp.float32),
        pltpu.VMEM((1, HG, T), jnp.float32),
        pltpu.SemaphoreType.DMA,
        pltpu.SemaphoreType.DMA,
    ],
    compiler_params=pltpu.CompilerParams(needs_layout_passes=False),
)(_tec_body)


@jax.jit
def kernel(relative_position_bias_table, relative_position_index):
  tbl = relative_position_bias_table.astype(jnp.float32)
  tbl_flat = jnp.pad(tbl, ((0, KPAD - K), (0, 0))).T.reshape(-1)
  idx = relative_position_index.astype(jnp.int32)
  out3 = _rpb_call(tbl_flat, idx)
  return jnp.transpose(out3, (1, 0, 2))
